# Initial kernel scaffold; baseline (speedup 1.0000x reference)
import jax
import jax.numpy as jnp
from jax.experimental import pallas as pl


def kernel(x, edge_index, W1, b1, W2, b2, Wd, bd):
    raise NotImplementedError("write your pallas kernel here")


# speedup vs baseline: 18.7276x; 18.7276x over previous
import functools

import jax
import jax.numpy as jnp
from jax import lax
from jax.experimental import pallas as pl
from jax.experimental.pallas import tpu as pltpu
from jax.experimental.pallas import tpu_sc as plsc

N_NODES = 10000
N_PAD = 10240
N_EDGES = 320000
NW = 32
N_CHUNK = 80
CHUNK = 125
ROWS_PER_TILE = N_PAD // 16
DEG_W = 128


def _sc_edge_agg(table, src3, dst3, zeros_hbm, feat):
    mesh = plsc.VectorSubcoreMesh(core_axis_name="c", subcore_axis_name="s")

    @functools.partial(
        pl.kernel,
        out_type=jax.ShapeDtypeStruct((2, N_PAD, feat), jnp.float32),
        mesh=mesh,
        scratch_types=[
            pltpu.VMEM((N_CHUNK, CHUNK), jnp.int32),
            pltpu.VMEM((N_CHUNK, CHUNK), jnp.int32),
            pltpu.VMEM((CHUNK, feat), jnp.float32),
            pltpu.VMEM_SHARED((N_PAD, feat), jnp.float32),
            pltpu.SemaphoreType.DMA,
        ],
    )
    def k(table_h, src_h, dst_h, zeros_h, out_h, src_v, dst_v, rows_v, acc_sh, sem):
        c = lax.axis_index("c")
        s = lax.axis_index("s")
        w = c * 16 + s
        pltpu.sync_copy(src_h.at[w], src_v)
        pltpu.sync_copy(dst_h.at[w], dst_v)
        pltpu.sync_copy(zeros_h, acc_sh.at[pl.ds(s * ROWS_PER_TILE, ROWS_PER_TILE)])
        plsc.subcore_barrier()

        def body(j, carry):
            pltpu.async_copy(table_h.at[src_v.at[j]], rows_v, sem).wait()
            pltpu.sync_copy(rows_v, acc_sh.at[dst_v.at[j]], add=True)
            return carry

        lax.fori_loop(0, N_CHUNK, body, 0)
        plsc.subcore_barrier()
        sl = pl.ds(s * ROWS_PER_TILE, ROWS_PER_TILE)
        pltpu.sync_copy(acc_sh.at[sl], out_h.at[c].at[sl])

    return k(table, src3, dst3, zeros_hbm)


def _sc_degree(dst3, ones_hbm, zeros_hbm):
    mesh = plsc.VectorSubcoreMesh(core_axis_name="c", subcore_axis_name="s")

    @functools.partial(
        pl.kernel,
        out_type=jax.ShapeDtypeStruct((2, N_PAD, DEG_W), jnp.float32),
        mesh=mesh,
        scratch_types=[
            pltpu.VMEM((N_CHUNK, CHUNK), jnp.int32),
            pltpu.VMEM((CHUNK, DEG_W), jnp.float32),
            pltpu.VMEM_SHARED((N_PAD, DEG_W), jnp.float32),
        ],
    )
    def k(dst_h, ones_h, zeros_h, out_h, dst_v, ones_v, acc_sh):
        c = lax.axis_index("c")
        s = lax.axis_index("s")
        w = c * 16 + s
        pltpu.sync_copy(dst_h.at[w], dst_v)
        pltpu.sync_copy(ones_h, ones_v)
        pltpu.sync_copy(zeros_h, acc_sh.at[pl.ds(s * ROWS_PER_TILE, ROWS_PER_TILE)])
        plsc.subcore_barrier()

        def body(j, carry):
            pltpu.sync_copy(ones_v, acc_sh.at[dst_v.at[j]], add=True)
            return carry

        lax.fori_loop(0, N_CHUNK, body, 0)
        plsc.subcore_barrier()
        sl = pl.ds(s * ROWS_PER_TILE, ROWS_PER_TILE)
        pltpu.sync_copy(acc_sh.at[sl], out_h.at[c].at[sl])

    return k(dst3, ones_hbm, zeros_hbm)


_R = 1000


def _tc_encode1(x, W1, p0, p1):
    def body(x_r, w_r, p0_r, p1_r, h_r, dis_r):
        deg = 1.0 + p0_r[...] + p1_r[...]
        dis = lax.rsqrt(deg)
        h = jnp.dot(x_r[...], w_r[...], preferred_element_type=jnp.float32,
                    precision=lax.Precision.HIGHEST)
        h_r[...] = h * dis
        dis_r[...] = dis

    return pl.pallas_call(
        body,
        grid=(N_NODES // _R,),
        in_specs=[
            pl.BlockSpec((_R, 128), lambda i: (i, 0)),
            pl.BlockSpec((128, 128), lambda i: (0, 0)),
            pl.BlockSpec((_R, 1), lambda i: (i, 0)),
            pl.BlockSpec((_R, 1), lambda i: (i, 0)),
        ],
        out_specs=[
            pl.BlockSpec((_R, 128), lambda i: (i, 0)),
            pl.BlockSpec((_R, 1), lambda i: (i, 0)),
        ],
        out_shape=[
            jax.ShapeDtypeStruct((N_NODES, 128), jnp.float32),
            jax.ShapeDtypeStruct((N_NODES, 1), jnp.float32),
        ],
    )(x, W1, p0, p1)


def _tc_encode2(a0, a1, h1p, dis, b1, W2):
    def body(a0_r, a1_r, h1p_r, dis_r, b1_r, w2_r, o_r):
        t = (a0_r[...] + a1_r[...] + h1p_r[...]) * dis_r[...] + b1_r[...]
        o = jnp.maximum(t, 0.0)
        h2 = jnp.dot(o, w2_r[...], preferred_element_type=jnp.float32,
                     precision=lax.Precision.HIGHEST)
        o_r[...] = jnp.concatenate(
            [h2 * dis_r[...], jnp.zeros((h2.shape[0], 64), jnp.float32)], axis=1)

    return pl.pallas_call(
        body,
        grid=(N_NODES // _R,),
        in_specs=[
            pl.BlockSpec((_R, 128), lambda i: (i, 0)),
            pl.BlockSpec((_R, 128), lambda i: (i, 0)),
            pl.BlockSpec((_R, 128), lambda i: (i, 0)),
            pl.BlockSpec((_R, 1), lambda i: (i, 0)),
            pl.BlockSpec((1, 128), lambda i: (0, 0)),
            pl.BlockSpec((128, 64), lambda i: (0, 0)),
        ],
        out_specs=pl.BlockSpec((_R, 128), lambda i: (i, 0)),
        out_shape=jax.ShapeDtypeStruct((N_NODES, 128), jnp.float32),
    )(a0, a1, h1p, dis, b1, W2)


def _tc_decode(a0, a1, h2p, dis, b2, Wd, bd):
    def body(a0_r, a1_r, h2p_r, dis_r, b2_r, wd_r, bd_r, o_r):
        z = (a0_r[...] + a1_r[...] + h2p_r[...]) * dis_r[...] + b2_r[...]
        dec = jnp.dot(z, wd_r[...], preferred_element_type=jnp.float32,
                      precision=lax.Precision.HIGHEST) + bd_r[...]
        o_r[...] = 1.0 / (1.0 + jnp.exp(-dec))

    return pl.pallas_call(
        body,
        grid=(N_NODES // _R,),
        in_specs=[
            pl.BlockSpec((_R, 128), lambda i: (i, 0)),
            pl.BlockSpec((_R, 128), lambda i: (i, 0)),
            pl.BlockSpec((_R, 128), lambda i: (i, 0)),
            pl.BlockSpec((_R, 1), lambda i: (i, 0)),
            pl.BlockSpec((1, 128), lambda i: (0, 0)),
            pl.BlockSpec((128, 128), lambda i: (0, 0)),
            pl.BlockSpec((1, 128), lambda i: (0, 0)),
        ],
        out_specs=pl.BlockSpec((_R, 128), lambda i: (i, 0)),
        out_shape=jax.ShapeDtypeStruct((N_NODES, 128), jnp.float32),
    )(a0, a1, h2p, dis, b2, Wd, bd)


def kernel(x, edge_index, W1, b1, W2, b2, Wd, bd):
    src3 = edge_index[0].reshape(NW, N_CHUNK, CHUNK)
    dst3 = edge_index[1].reshape(NW, N_CHUNK, CHUNK)

    zeros_deg = jnp.zeros((ROWS_PER_TILE, DEG_W), jnp.float32)
    ones_deg = jnp.ones((CHUNK, DEG_W), jnp.float32)
    degp = _sc_degree(dst3, ones_deg, zeros_deg)
    p0 = degp[0, :N_NODES, 0:1]
    p1 = degp[1, :N_NODES, 0:1]

    h1p, dis = _tc_encode1(x, W1, p0, p1)

    zeros128 = jnp.zeros((ROWS_PER_TILE, 128), jnp.float32)
    agg1 = _sc_edge_agg(h1p, src3, dst3, zeros128, 128)
    h2p = _tc_encode2(agg1[0, :N_NODES], agg1[1, :N_NODES], h1p, dis,
                      b1.reshape(1, 128), W2)

    agg2 = _sc_edge_agg(h2p, src3, dst3, zeros128, 128)
    b2p = jnp.zeros((1, 128), jnp.float32).at[0, :64].set(b2)
    Wdp = jnp.zeros((128, 128), jnp.float32).at[:64, :].set(Wd)
    return _tc_decode(agg2[0, :N_NODES], agg2[1, :N_NODES], h2p, dis,
                      b2p, Wdp, bd.reshape(1, 128))
